# baseline (speedup 1.0000x reference)
import jax
import jax.numpy as jnp
from jax.experimental import pallas as pl
from jax.experimental.pallas import tpu as pltpu
from jax.experimental.pallas import tpu_sc as plsc


_WB = 64
_LANES = 16


def kernel(inputs, row_embeddings, col_embeddings):
    b, h, w, c = inputs.shape
    wb = _WB
    mesh = plsc.VectorSubcoreMesh(core_axis_name="core", subcore_axis_name="subcore")

    @pl.kernel(
        out_type=jax.ShapeDtypeStruct((b, h, w, c), inputs.dtype),
        mesh=mesh,
        scratch_types=[],
    )
    def sc_kernel(x_hbm, row_hbm, col_hbm, o_hbm):
        def body(x_vmem, row_vmem, col_vmem, o_vmem):
            rvs = [
                row_vmem.at[0, pl.ds(cc, _LANES)][...]
                for cc in range(0, c, _LANES)
            ]

            @plsc.parallel_loop(0, wb, unroll=8)
            def _(wr):
                for j, cc in enumerate(range(0, c, _LANES)):
                    cv = col_vmem.at[wr, pl.ds(cc, _LANES)][...]
                    xv = x_vmem.at[0, 0, wr, pl.ds(cc, _LANES)][...]
                    o_vmem.at[0, 0, wr, pl.ds(cc, _LANES)][...] = xv + rvs[j] + cv

        pltpu.emit_pipeline(
            body,
            grid=(b, h, w // wb),
            in_specs=[
                pl.BlockSpec((1, 1, wb, c), index_map=lambda bi, hi, wi: (bi, hi, wi, 0)),
                pl.BlockSpec((1, c), index_map=lambda bi, hi, wi: (hi, 0)),
                pl.BlockSpec((wb, c), index_map=lambda bi, hi, wi: (wi, 0)),
            ],
            out_specs=[
                pl.BlockSpec((1, 1, wb, c), index_map=lambda bi, hi, wi: (bi, hi, wi, 0)),
            ],
            core_axis_name=("core", "subcore"),
            dimension_semantics=(pltpu.PARALLEL, pltpu.PARALLEL, pltpu.PARALLEL),
        )(x_hbm, row_hbm, col_hbm, o_hbm)

    return sc_kernel(inputs, row_embeddings, col_embeddings)
